# baseline (speedup 1.0000x reference)
import functools

import jax
import jax.numpy as jnp
from jax import lax
from jax.experimental import pallas as pl
from jax.experimental.pallas import tpu as pltpu
from jax.experimental.pallas import tpu_sc as plsc

NUM_EDGES = 320000
DIM = 128
L = 16
BLK = 128

NC = 2
NS = 16
NW = NC * NS
NBLK = NUM_EDGES // BLK
CB = 2
CHUNK = CB * BLK
NCHUNKS = NBLK // CB
NGROUP = CHUNK // L
NBT = 10
NST = 7
NROWS = NBT * NST
NCG = DIM // L


@functools.partial(
    pl.kernel,
    out_type=jax.ShapeDtypeStruct((NUM_EDGES, DIM), jnp.float32),
    mesh=plsc.VectorSubcoreMesh(core_axis_name="c", subcore_axis_name="s"),
    compiler_params=pltpu.CompilerParams(needs_layout_passes=False),
    scratch_types=[
        pltpu.VMEM((3, CB, 4, BLK), jnp.float32),
        pltpu.VMEM((3, CHUNK, DIM), jnp.float32),
        pltpu.VMEM((NROWS * DIM,), jnp.float32),
        pltpu.VMEM((NBT, DIM), jnp.float32),
        pltpu.VMEM((NST, DIM), jnp.float32),
        pltpu.VMEM((2, DIM), jnp.float32),
        pltpu.VMEM((DIM,), jnp.float32),
        pltpu.SemaphoreType.DMA,
        pltpu.SemaphoreType.DMA,
        pltpu.SemaphoreType.DMA,
        pltpu.SemaphoreType.DMA,
        pltpu.SemaphoreType.DMA,
        pltpu.SemaphoreType.DMA,
    ],
)
def _sc_embed(feat_hbm, btab_hbm, stab_hbm, w_hbm, b_hbm, out_hbm,
              feat_v, out_v, tab_v, btab_v, stab_v, w_v, b_v,
              sem_f0, sem_f1, sem_f2, sem_o0, sem_o1, sem_o2):
    wid = lax.axis_index("s") * NC + lax.axis_index("c")

    pltpu.sync_copy(btab_hbm, btab_v)
    pltpu.sync_copy(stab_hbm, stab_v)
    pltpu.sync_copy(w_hbm, w_v)
    pltpu.sync_copy(b_hbm, b_v)

    iota = lax.iota(jnp.int32, L)
    tab_cg = [tab_v.at[pl.ds(cg * L, NROWS * DIM - (NCG - 1) * L)]
              for cg in range(NCG)]

    w0s = [w_v[0, pl.ds(cg * L, L)] for cg in range(NCG)]
    w1s = [w_v[1, pl.ds(cg * L, L)] for cg in range(NCG)]

    def build_row(i, carry):
        for cg in range(NCG):
            sl = pl.ds(cg * L, L)
            base = btab_v[i, sl] + b_v[sl]
            for j in range(NST):
                tab_v[pl.ds((i * NST + j) * DIM + cg * L, L)] = base + stab_v[j, sl]
        return carry
    lax.fori_loop(0, NBT, build_row, 0)

    n_mine = NCHUNKS // NW + jnp.where(wid < NCHUNKS % NW, 1, 0)
    sems_f = [sem_f0, sem_f1, sem_f2]
    sems_o = [sem_o0, sem_o1, sem_o2]
    NBUF = 3

    def feat_load(k, b):
        ci = wid + k * NW
        pltpu.async_copy(feat_hbm.at[pl.ds(ci * CB, CB)], feat_v.at[b],
                         sems_f[b])

    def compute_chunk(k, b):
        def group_body(g, gcarry):
            gb = g * L
            bi = g // (BLK // L)
            lo = (g % (BLK // L)) * L
            f0v = feat_v[b, bi, 0, pl.ds(lo, L)]
            f1v = feat_v[b, bi, 1, pl.ds(lo, L)]
            f2v = feat_v[b, bi, 2, pl.ds(lo, L)]
            f3v = feat_v[b, bi, 3, pl.ds(lo, L)]
            btv = jnp.clip((f0v * 2.0).astype(jnp.int32), 0, NBT - 1)
            stv = jnp.clip(f3v.astype(jnp.int32), 0, NST - 1)
            rbasev = (btv * NST + stv) * DIM
            for e in range(L):
                rbase = jnp.full((L,), rbasev[e]) + iota
                f1 = jnp.full((L,), f1v[e])
                f2 = jnp.full((L,), f2v[e])
                tgs = [plsc.load_gather(tab_cg[cg], [rbase]) for cg in range(NCG)]
                for cg in range(NCG):
                    out_v[b, gb + e, pl.ds(cg * L, L)] = (
                        (tgs[cg] + f1 * w0s[cg]) + f2 * w1s[cg])
            return gcarry
        lax.fori_loop(0, NGROUP, group_body, 0)

    def wait_feat(b):
        pltpu.make_async_copy(feat_hbm.at[pl.ds(0, CB)], feat_v.at[b],
                              sems_f[b]).wait()

    def wait_out(b):
        pltpu.make_async_copy(out_v.at[b], out_hbm.at[pl.ds(0, CHUNK)],
                              sems_o[b]).wait()

    for b in range(NBUF):
        feat_load(b, b)

    def ring_body(kr, carry):
        for b in range(NBUF):
            k = kr * NBUF + b

            @pl.when(k < n_mine)
            def _():
                wait_feat(b)

                @pl.when(k >= NBUF)
                def _():
                    wait_out(b)

                compute_chunk(k, b)
                ci = wid + k * NW
                pltpu.async_copy(out_v.at[b], out_hbm.at[pl.ds(ci * CHUNK, CHUNK)],
                                 sems_o[b])

                @pl.when(k + NBUF < n_mine)
                def _():
                    feat_load(k + NBUF, b)
        return carry
    lax.fori_loop(0, (NCHUNKS // NW + NBUF) // NBUF, ring_body, 0)

    for b in range(NBUF):
        wait_out(b)


def kernel(edge_features, bond_type_table, stereo_table, W_binary, b_binary):
    feat3 = edge_features.reshape(NBLK, BLK, 4).transpose(0, 2, 1)
    return _sc_embed(feat3, bond_type_table, stereo_table,
                     W_binary.T, b_binary)


# speedup vs baseline: 1.0610x; 1.0610x over previous
import functools

import jax
import jax.numpy as jnp
from jax import lax
from jax.experimental import pallas as pl
from jax.experimental.pallas import tpu as pltpu
from jax.experimental.pallas import tpu_sc as plsc

NUM_EDGES = 320000
DIM = 128
L = 16
BLK = 128

NC = 2
NS = 16
NW = NC * NS
NBLK = NUM_EDGES // BLK
CB = 2
CHUNK = CB * BLK
NCHUNKS = NBLK // CB
NGROUP = CHUNK // L
NBT = 10
NST = 7
NROWS = NBT * NST
NCG = DIM // L


@functools.partial(
    pl.kernel,
    out_type=jax.ShapeDtypeStruct((NUM_EDGES, DIM), jnp.float32),
    mesh=plsc.VectorSubcoreMesh(core_axis_name="c", subcore_axis_name="s"),
    compiler_params=pltpu.CompilerParams(needs_layout_passes=False),
    scratch_types=[
        pltpu.VMEM((2, CB, 4, BLK), jnp.float32),
        pltpu.VMEM((2, CHUNK, DIM), jnp.float32),
        pltpu.VMEM((NROWS * DIM,), jnp.float32),
        pltpu.VMEM((NBT, DIM), jnp.float32),
        pltpu.VMEM((NST, DIM), jnp.float32),
        pltpu.VMEM((2, DIM), jnp.float32),
        pltpu.VMEM((DIM,), jnp.float32),
        pltpu.SemaphoreType.DMA,
        pltpu.SemaphoreType.DMA,
        pltpu.SemaphoreType.DMA,
        pltpu.SemaphoreType.DMA,
    ],
)
def _sc_embed(feat_hbm, btab_hbm, stab_hbm, w_hbm, b_hbm, out_hbm,
              feat_v, out_v, tab_v, btab_v, stab_v, w_v, b_v,
              sem_f0, sem_f1, sem_o0, sem_o1):
    wid = lax.axis_index("s") * NC + lax.axis_index("c")

    pltpu.sync_copy(btab_hbm, btab_v)
    pltpu.sync_copy(stab_hbm, stab_v)
    pltpu.sync_copy(w_hbm, w_v)
    pltpu.sync_copy(b_hbm, b_v)

    iota = lax.iota(jnp.int32, L)
    tab_cg = [tab_v.at[pl.ds(cg * L, NROWS * DIM - (NCG - 1) * L)]
              for cg in range(NCG)]

    w0s = [w_v[0, pl.ds(cg * L, L)] for cg in range(NCG)]
    w1s = [w_v[1, pl.ds(cg * L, L)] for cg in range(NCG)]

    def build_row(i, carry):
        for cg in range(NCG):
            sl = pl.ds(cg * L, L)
            base = btab_v[i, sl] + b_v[sl]
            for j in range(NST):
                tab_v[pl.ds((i * NST + j) * DIM + cg * L, L)] = base + stab_v[j, sl]
        return carry
    lax.fori_loop(0, NBT, build_row, 0)

    n_mine = NCHUNKS // NW + jnp.where(wid < NCHUNKS % NW, 1, 0)
    sems_f = [sem_f0, sem_f1]
    sems_o = [sem_o0, sem_o1]
    NBUF = 2

    def feat_load(k, b):
        ci = wid + k * NW
        pltpu.async_copy(feat_hbm.at[pl.ds(ci * CB, CB)], feat_v.at[b],
                         sems_f[b])

    def compute_chunk(k, b):
        def group_body(g, gcarry):
            gb = g * L
            bi = g // (BLK // L)
            lo = (g % (BLK // L)) * L
            f0v = feat_v[b, bi, 0, pl.ds(lo, L)]
            f1v = feat_v[b, bi, 1, pl.ds(lo, L)]
            f2v = feat_v[b, bi, 2, pl.ds(lo, L)]
            f3v = feat_v[b, bi, 3, pl.ds(lo, L)]
            btv = jnp.clip((f0v * 2.0).astype(jnp.int32), 0, NBT - 1)
            stv = jnp.clip(f3v.astype(jnp.int32), 0, NST - 1)
            rbasev = (btv * NST + stv) * DIM
            for ep in range(L // 2):
                e0, e1 = 2 * ep, 2 * ep + 1
                rb0 = jnp.full((L,), rbasev[e0]) + iota
                rb1 = jnp.full((L,), rbasev[e1]) + iota
                f1a = jnp.full((L,), f1v[e0])
                f2a = jnp.full((L,), f2v[e0])
                f1b_ = jnp.full((L,), f1v[e1])
                f2b_ = jnp.full((L,), f2v[e1])
                tg0 = [plsc.load_gather(tab_cg[cg], [rb0]) for cg in range(NCG)]
                tg1 = [plsc.load_gather(tab_cg[cg], [rb1]) for cg in range(NCG)]
                for cg in range(NCG):
                    out_v[b, gb + e0, pl.ds(cg * L, L)] = (
                        (tg0[cg] + f1a * w0s[cg]) + f2a * w1s[cg])
                    out_v[b, gb + e1, pl.ds(cg * L, L)] = (
                        (tg1[cg] + f1b_ * w0s[cg]) + f2b_ * w1s[cg])
            return gcarry
        lax.fori_loop(0, NGROUP, group_body, 0)

    def wait_feat(b):
        pltpu.make_async_copy(feat_hbm.at[pl.ds(0, CB)], feat_v.at[b],
                              sems_f[b]).wait()

    def wait_out(b):
        pltpu.make_async_copy(out_v.at[b], out_hbm.at[pl.ds(0, CHUNK)],
                              sems_o[b]).wait()

    for b in range(NBUF):
        feat_load(b, b)

    def ring_body(kr, carry):
        for b in range(NBUF):
            k = kr * NBUF + b

            @pl.when(k < n_mine)
            def _():
                wait_feat(b)

                @pl.when(k >= NBUF)
                def _():
                    wait_out(b)

                compute_chunk(k, b)
                ci = wid + k * NW
                pltpu.async_copy(out_v.at[b], out_hbm.at[pl.ds(ci * CHUNK, CHUNK)],
                                 sems_o[b])

                @pl.when(k + NBUF < n_mine)
                def _():
                    feat_load(k + NBUF, b)
        return carry
    lax.fori_loop(0, (NCHUNKS // NW + NBUF) // NBUF, ring_body, 0)

    for b in range(NBUF):
        wait_out(b)


def kernel(edge_features, bond_type_table, stereo_table, W_binary, b_binary):
    feat3 = edge_features.reshape(NBLK, BLK, 4).transpose(0, 2, 1)
    return _sc_embed(feat3, bond_type_table, stereo_table,
                     W_binary.T, b_binary)
